# Initial kernel scaffold; baseline (speedup 1.0000x reference)
import jax
import jax.numpy as jnp
from jax.experimental import pallas as pl


def kernel(x, edge_index, edge_attr, batch, Wr0, br0, Ws0, Wr1, br1, Ws1, Wr2, br2, Ws2, Wr3, br3, Ws3, Wr4, br4, Ws4, MW0, Mb0, MW1, Mb1, MW2, Mb2):
    raise NotImplementedError("write your pallas kernel here")


# speedup vs baseline: 1.2017x; 1.2017x over previous
import functools

import jax
import jax.numpy as jnp
from jax import lax
from jax.experimental import pallas as pl
from jax.experimental.pallas import tpu as pltpu
from jax.experimental.pallas import tpu_sc as plsc

F32 = jnp.float32
NNODE = 10000
NP = 10240
BN = 512
NBLK = NP // BN
NGRAPH = 64


def _layer_body(agg_ref, h_ref, wr_ref, ws_ref, br_ref, out_ref):
    acc = jnp.dot(agg_ref[...], wr_ref[...], preferred_element_type=F32)
    acc += jnp.dot(h_ref[...], ws_ref[...], preferred_element_type=F32)
    out_ref[...] = jnp.maximum(acc + br_ref[...], 0.0)


def _layer_full(agg, h, Wr, Ws, br2d):
    cin, cout = Wr.shape
    return pl.pallas_call(
        _layer_body,
        grid=(NBLK,),
        in_specs=[
            pl.BlockSpec((BN, cin), lambda i: (i, 0)),
            pl.BlockSpec((BN, cin), lambda i: (i, 0)),
            pl.BlockSpec((cin, cout), lambda i: (0, 0)),
            pl.BlockSpec((cin, cout), lambda i: (0, 0)),
            pl.BlockSpec((1, cout), lambda i: (0, 0)),
        ],
        out_specs=pl.BlockSpec((BN, cout), lambda i: (i, 0)),
        out_shape=jax.ShapeDtypeStruct((NP, cout), F32),
    )(agg, h, Wr, Ws, br2d)


def _pre_body(a_ref, b_ref, wr_ref, ws_ref, br_ref, t_ref, r_ref):
    h = jnp.maximum(a_ref[...] + b_ref[...], 0.0)
    t_ref[...] = jnp.dot(h, wr_ref[...], preferred_element_type=F32)
    r_ref[...] = jnp.dot(h, ws_ref[...], preferred_element_type=F32) + br_ref[...]


def _pre(a, b, Wr, Ws, br2d):
    cin, cout = Wr.shape
    return pl.pallas_call(
        _pre_body,
        grid=(NBLK,),
        in_specs=[
            pl.BlockSpec((BN, cin), lambda i: (i, 0)),
            pl.BlockSpec((BN, cin), lambda i: (i, 0)),
            pl.BlockSpec((cin, cout), lambda i: (0, 0)),
            pl.BlockSpec((cin, cout), lambda i: (0, 0)),
            pl.BlockSpec((1, cout), lambda i: (0, 0)),
        ],
        out_specs=[
            pl.BlockSpec((BN, cout), lambda i: (i, 0)),
            pl.BlockSpec((BN, cout), lambda i: (i, 0)),
        ],
        out_shape=[
            jax.ShapeDtypeStruct((NP, cout), F32),
            jax.ShapeDtypeStruct((NP, cout), F32),
        ],
    )(a, b, Wr, Ws, br2d)


def _pool_body(a_ref, b_ref, batch_ref, sums_ref, cnts_ref):
    i = pl.program_id(0)

    @pl.when(i == 0)
    def _():
        sums_ref[...] = jnp.zeros_like(sums_ref)
        cnts_ref[...] = jnp.zeros_like(cnts_ref)

    h = jnp.maximum(a_ref[...] + b_ref[...], 0.0)
    bids = batch_ref[...]
    valid = (bids < NGRAPH).astype(F32)
    onehot = jnp.where(
        lax.broadcasted_iota(jnp.int32, (NGRAPH, BN), 0) == bids, 1.0, 0.0
    ).astype(F32)
    h = h * valid.reshape(BN, 1)
    sums_ref[...] += jnp.dot(onehot, h, preferred_element_type=F32)
    cnts_ref[...] += jnp.dot(
        onehot, jnp.broadcast_to(valid.reshape(BN, 1), (BN, 128)),
        preferred_element_type=F32)


def _pool(a, b, batch2d):
    return pl.pallas_call(
        _pool_body,
        grid=(NBLK,),
        in_specs=[
            pl.BlockSpec((BN, 128), lambda i: (i, 0)),
            pl.BlockSpec((BN, 128), lambda i: (i, 0)),
            pl.BlockSpec((1, BN), lambda i: (0, i)),
        ],
        out_specs=[
            pl.BlockSpec((NGRAPH, 128), lambda i: (0, 0)),
            pl.BlockSpec((NGRAPH, 128), lambda i: (0, 0)),
        ],
        out_shape=[
            jax.ShapeDtypeStruct((NGRAPH, 128), F32),
            jax.ShapeDtypeStruct((NGRAPH, 128), F32),
        ],
    )(a, b, batch2d)


def _head_body(sums_ref, cnts_ref, w0_ref, b0_ref, w1_ref, b1_ref, w2_ref,
               b2_ref, out_ref):
    g = sums_ref[...] / jnp.maximum(cnts_ref[...], 1.0)
    g = jnp.maximum(jnp.dot(g, w0_ref[...], preferred_element_type=F32)
                    + b0_ref[...], 0.0)
    g = jnp.maximum(jnp.dot(g, w1_ref[...], preferred_element_type=F32)
                    + b1_ref[...], 0.0)
    out_ref[...] = jnp.dot(g, w2_ref[...], preferred_element_type=F32) + b2_ref[...]


def _head(sums, cnts, MW0, Mb0, MW1p, Mb1p, MW2p, Mb2p):
    return pl.pallas_call(
        _head_body,
        out_shape=jax.ShapeDtypeStruct((NGRAPH, 128), F32),
    )(sums, cnts, MW0, Mb0, MW1p, Mb1p, MW2p, Mb2p)


def _aggregate(h, src, dst, w):
    msg = h[src] * w[:, None]
    return jax.ops.segment_sum(msg, dst, num_segments=NP)


def kernel(x, edge_index, edge_attr, batch, Wr0, br0, Ws0, Wr1, br1, Ws1,
           Wr2, br2, Ws2, Wr3, br3, Ws3, Wr4, br4, Ws4, MW0, Mb0, MW1, Mb1,
           MW2, Mb2):
    src, dst = edge_index[0], edge_index[1]
    x_pad = jnp.pad(x, ((0, NP - NNODE), (0, 0)))
    batch_pad = jnp.pad(batch, (0, NP - NNODE), constant_values=NGRAPH)
    batch2d = batch_pad.reshape(1, NP)

    h = x_pad
    for Wr, br, Ws in ((Wr0, br0, Ws0), (Wr1, br1, Ws1), (Wr2, br2, Ws2)):
        agg = _aggregate(h, src, dst, edge_attr)
        h = _layer_full(agg, h, Wr, Ws, br.reshape(1, -1))

    zeros3 = jnp.zeros_like(h)
    t3, r3 = _pre(h, zeros3, Wr3, Ws3, br3.reshape(1, -1))
    aggw3 = _aggregate(t3, src, dst, edge_attr)
    t4, r4 = _pre(aggw3, r3, Wr4, Ws4, br4.reshape(1, -1))
    aggw4 = _aggregate(t4, src, dst, edge_attr)

    sums, cnts = _pool(aggw4, r4, batch2d)
    MW1p = jnp.pad(MW1, ((0, 0), (0, 64)))
    Mb1p = jnp.pad(Mb1, (0, 64)).reshape(1, 128)
    MW2p = jnp.pad(MW2, ((0, 64), (0, 127)))
    Mb2p = jnp.pad(Mb2, (0, 127)).reshape(1, 128)
    out = _head(sums, cnts, MW0, Mb0.reshape(1, 128), MW1p, Mb1p, MW2p, Mb2p)
    return out[:, :1]
